# baseline (speedup 1.0000x reference)
import functools

import jax
import jax.numpy as jnp
from jax import lax
from jax.experimental import pallas as pl
from jax.experimental.pallas import tpu as pltpu
from jax.experimental.pallas import tpu_sc as plsc

N, K, H = 10000, 32, 128
NC, NS = 2, 16
NW = NC * NS
NPW = 320
NPAD = NW * NPW
CH = 2
NCHUNK = NPW // CH
HV = H // 16
RB = 1000
RPS = NPAD // NS
SLAB = 32


def _tc1_body(x_ref, w1t_ref, w1b_ref, u1t_ref, b1_ref, cb_ref,
              a_ref, b_ref, t1_ref):
    x = x_ref[...]
    a_ref[...] = jnp.dot(x, w1t_ref[...], preferred_element_type=jnp.float32) + b1_ref[...]
    b_ref[...] = jnp.dot(x, w1b_ref[...], preferred_element_type=jnp.float32)
    t1_ref[...] = jnp.dot(x, u1t_ref[...], preferred_element_type=jnp.float32) + cb_ref[...]


def _tc2_body(t1_ref, s_ref, x_ref, lb_ref, wm_ref, w2_ref, b2_ref, g_ref, be_ref,
              o_ref):
    h = t1_ref[...] + jnp.dot(s_ref[...], wm_ref[...], preferred_element_type=jnp.float32)
    h = h / (1.0 + jnp.exp(-h))
    u = jnp.dot(h, w2_ref[...], preferred_element_type=jnp.float32) + b2_ref[...]
    y = x_ref[...] + u + lb_ref[...]
    mu = jnp.mean(y, axis=1, keepdims=True)
    yc = y - mu
    var = jnp.mean(yc * yc, axis=1, keepdims=True)
    o_ref[...] = yc * lax.rsqrt(var + 1e-5) * g_ref[...] + be_ref[...]


_row_spec = pl.BlockSpec((RB, H), lambda i: (i, 0))
_w_spec = pl.BlockSpec((H, H), lambda i: (0, 0))
_b_spec = pl.BlockSpec((1, H), lambda i: (0, 0))

_tc1 = pl.pallas_call(
    _tc1_body,
    grid=(N // RB,),
    in_specs=[_row_spec, _w_spec, _w_spec, _w_spec, _b_spec, _b_spec],
    out_specs=[_row_spec, _row_spec, _row_spec],
    out_shape=[jax.ShapeDtypeStruct((NPAD, H), jnp.float32)] * 3,
)

_tc2 = pl.pallas_call(
    _tc2_body,
    grid=(N // RB,),
    in_specs=[_row_spec, _row_spec, _row_spec, _row_spec,
              _w_spec, _w_spec, _b_spec, _b_spec, _b_spec],
    out_specs=_row_spec,
    out_shape=jax.ShapeDtypeStruct((N, H), jnp.float32),
)

_sc_mesh = plsc.VectorSubcoreMesh(core_axis_name="c", subcore_axis_name="s")


@functools.partial(
    pl.kernel,
    out_type=jax.ShapeDtypeStruct((NPAD, H), jnp.float32),
    mesh=_sc_mesh,
    scratch_types=[
        pltpu.VMEM((NCHUNK, CH * K), jnp.int32),
        pltpu.VMEM((SLAB, H), jnp.float32),
        pltpu.VMEM((2, CH * K, H), jnp.float32),
        pltpu.VMEM((2, CH, H), jnp.float32),
        pltpu.VMEM_SHARED((NPAD, H), jnp.float32),
        pltpu.SemaphoreType.DMA,
        pltpu.SemaphoreType.DMA,
        pltpu.SemaphoreType.DMA,
        pltpu.SemaphoreType.DMA,
        pltpu.SemaphoreType.DMA,
        pltpu.SemaphoreType.DMA,
    ],
)
def _sc_gather_silu_sum(a_hbm, b_hbm, idx_hbm, out_hbm,
                        idx_v, a_v, rows_v, acc_v, b_sp,
                        gsem0a, gsem0b, gsem1a, gsem1b, osem0, osem1):
    sub = lax.axis_index("s")
    wid = sub * NC + lax.axis_index("c")
    base = wid * NPW

    pltpu.sync_copy(b_hbm.at[pl.ds(sub * RPS, RPS)],
                    b_sp.at[pl.ds(sub * RPS, RPS)])
    pltpu.sync_copy(idx_hbm.at[wid], idx_v)
    pltpu.sync_copy(a_hbm.at[pl.ds(base, SLAB)], a_v)
    plsc.subcore_barrier()

    gsems = ((gsem0a, gsem0b), (gsem1a, gsem1b))
    osems = (osem0, osem1)
    HK = CH * K // 2

    def start_gather(c, b):
        pltpu.async_copy(b_sp.at[idx_v.at[c, pl.ds(0, HK)]],
                         rows_v.at[b, pl.ds(0, HK)], gsems[b][0])
        pltpu.async_copy(b_sp.at[idx_v.at[c, pl.ds(HK, HK)]],
                         rows_v.at[b, pl.ds(HK, HK)], gsems[b][1])

    def wait_gather(c, b):
        pltpu.make_async_copy(b_sp.at[idx_v.at[c, pl.ds(0, HK)]],
                              rows_v.at[b, pl.ds(0, HK)], gsems[b][0]).wait()
        pltpu.make_async_copy(b_sp.at[idx_v.at[c, pl.ds(HK, HK)]],
                              rows_v.at[b, pl.ds(HK, HK)], gsems[b][1]).wait()

    start_gather(0, 0)

    def compute_chunk(c, rbuf, abuf):
        for n in range(CH):
            node = (c % (SLAB // CH)) * CH + n
            a_vecs = [a_v[node, pl.ds(16 * h, 16)] for h in range(HV)]

            def kbody(k, accs):
                out = list(accs)
                for dk in range(2):
                    r = n * K + 2 * k + dk
                    for h in range(HV):
                        row = rbuf[r, pl.ds(16 * h, 16)]
                        sv = a_vecs[h] + row
                        out[h] = out[h] + sv / (1.0 + jnp.exp(-sv))
                return tuple(out)

            accs = lax.fori_loop(
                0, K // 2, kbody,
                tuple(jnp.zeros((16,), jnp.float32) for _ in range(HV)))
            for h in range(HV):
                abuf[n, pl.ds(16 * h, 16)] = accs[h]

    def pair_fn(p, carry):
        for b in range(2):
            c = 2 * p + b

            @pl.when(jnp.logical_and(c % (SLAB // CH) == 0, c > 0))
            def _():
                pltpu.sync_copy(
                    a_hbm.at[pl.ds(base + (c // (SLAB // CH)) * SLAB, SLAB)],
                    a_v)

            @pl.when(c + 1 < NCHUNK)
            def _():
                start_gather(c + 1, 1 - b)

            wait_gather(c, b)

            @pl.when(c >= 2)
            def _():
                pltpu.make_async_copy(acc_v.at[b], out_hbm.at[pl.ds(0, CH)],
                                      osems[b]).wait()

            compute_chunk(c, rows_v.at[b], acc_v.at[b])
            pltpu.async_copy(acc_v.at[b], out_hbm.at[pl.ds(base + c * CH, CH)],
                             osems[b])
        return carry

    lax.fori_loop(0, NCHUNK // 2, pair_fn, 0)
    pltpu.make_async_copy(acc_v.at[0], out_hbm.at[pl.ds(0, CH)], osem0).wait()
    pltpu.make_async_copy(acc_v.at[1], out_hbm.at[pl.ds(0, CH)], osem1).wait()


def kernel(x, lat_bias, knn_idx, msg_W1, msg_b1, msg_W2, msg_b2,
           upd_W1, upd_b1, upd_W2, upd_b2, ln_gamma, ln_beta):
    x2 = x[0]
    idx3 = jnp.pad(knn_idx, ((0, NPAD - N), (0, 0))).reshape(NW, NCHUNK, CH * K)

    W1t, W1b = msg_W1[:H], msg_W1[H:]
    U1t, U1b = upd_W1[:H], upd_W1[H:]
    cb = msg_b2 @ U1b + upd_b1
    Wm = (msg_W2 / K) @ U1b

    A, B, T1 = _tc1(x2, W1t, W1b, U1t, msg_b1[None], cb[None])
    S = _sc_gather_silu_sum(A, B, idx3)
    out = _tc2(T1, S, x2, lat_bias[0], Wm, upd_W2, upd_b2[None],
               ln_gamma[None], ln_beta[None])
    return out[None]


# speedup vs baseline: 1.0344x; 1.0344x over previous
import functools

import jax
import jax.numpy as jnp
from jax import lax
from jax.experimental import pallas as pl
from jax.experimental.pallas import tpu as pltpu
from jax.experimental.pallas import tpu_sc as plsc

N, K, H = 10000, 32, 128
NC, NS = 2, 16
NW = NC * NS
NPW = 320
NPAD = NW * NPW
CH = 2
NCHUNK = NPW // CH
HV = H // 16
RB = 1000
RPS = NPAD // NS
SLAB = 64


def _tc1_body(x_ref, w1t_ref, w1b_ref, u1t_ref, u1b_ref, b1_ref, b2_ref, ub1_ref,
              a_ref, b_ref, t1_ref):
    x = x_ref[...]
    cb = jnp.dot(b2_ref[...], u1b_ref[...],
                 preferred_element_type=jnp.float32) + ub1_ref[...]
    a_ref[...] = jnp.dot(x, w1t_ref[...], preferred_element_type=jnp.float32) + b1_ref[...]
    b_ref[...] = jnp.dot(x, w1b_ref[...], preferred_element_type=jnp.float32)
    t1_ref[...] = jnp.dot(x, u1t_ref[...], preferred_element_type=jnp.float32) + cb


def _tc2_body(t1_ref, s_ref, x_ref, lb_ref, mw2_ref, u1b_ref, w2_ref, b2_ref, g_ref, be_ref,
              o_ref):
    wm = jnp.dot(mw2_ref[...], u1b_ref[...],
                 preferred_element_type=jnp.float32) * (1.0 / K)
    h = t1_ref[...] + jnp.dot(s_ref[...], wm, preferred_element_type=jnp.float32)
    h = h / (1.0 + jnp.exp(-h))
    u = jnp.dot(h, w2_ref[...], preferred_element_type=jnp.float32) + b2_ref[...]
    y = x_ref[...] + u + lb_ref[...]
    mu = jnp.mean(y, axis=1, keepdims=True)
    yc = y - mu
    var = jnp.mean(yc * yc, axis=1, keepdims=True)
    o_ref[...] = yc * lax.rsqrt(var + 1e-5) * g_ref[...] + be_ref[...]


_row_spec = pl.BlockSpec((RB, H), lambda i: (i, 0))
_w_spec = pl.BlockSpec((H, H), lambda i: (0, 0))
_wtop_spec = pl.BlockSpec((H, H), lambda i: (0, 0))
_wbot_spec = pl.BlockSpec((H, H), lambda i: (1, 0))
_b_spec = pl.BlockSpec((1, H), lambda i: (0, 0))

_tc1 = pl.pallas_call(
    _tc1_body,
    grid=(N // RB,),
    in_specs=[_row_spec, _wtop_spec, _wbot_spec, _wtop_spec, _wbot_spec,
              _b_spec, _b_spec, _b_spec],
    out_specs=[_row_spec, _row_spec, _row_spec],
    out_shape=[jax.ShapeDtypeStruct((NPAD, H), jnp.float32)] * 3,
)

_tc2 = pl.pallas_call(
    _tc2_body,
    grid=(N // RB,),
    in_specs=[_row_spec, _row_spec, _row_spec, _row_spec,
              _w_spec, _wbot_spec, _w_spec, _b_spec, _b_spec, _b_spec],
    out_specs=_row_spec,
    out_shape=jax.ShapeDtypeStruct((N, H), jnp.float32),
)

_sc_mesh = plsc.VectorSubcoreMesh(core_axis_name="c", subcore_axis_name="s")


@functools.partial(
    pl.kernel,
    out_type=jax.ShapeDtypeStruct((NPAD, H), jnp.float32),
    mesh=_sc_mesh,
    scratch_types=[
        pltpu.VMEM((NCHUNK, CH * K), jnp.int32),
        pltpu.VMEM((SLAB, H), jnp.float32),
        pltpu.VMEM((2, CH * K, H), jnp.float32),
        pltpu.VMEM((2, CH, H), jnp.float32),
        pltpu.VMEM_SHARED((NPAD, H), jnp.float32),
        pltpu.SemaphoreType.DMA,
        pltpu.SemaphoreType.DMA,
        pltpu.SemaphoreType.DMA,
        pltpu.SemaphoreType.DMA,
    ],
)
def _sc_gather_silu_sum(a_hbm, b_hbm, idx_hbm, out_hbm,
                        idx_v, a_v, rows_v, acc_v, b_sp,
                        gsem0, gsem1, osem0, osem1):
    sub = lax.axis_index("s")
    wid = sub * NC + lax.axis_index("c")
    base = wid * NPW

    pltpu.sync_copy(b_hbm.at[pl.ds(sub * RPS, RPS)],
                    b_sp.at[pl.ds(sub * RPS, RPS)])
    pltpu.sync_copy(idx_hbm.at[wid], idx_v)
    pltpu.sync_copy(a_hbm.at[pl.ds(base, SLAB)], a_v)
    plsc.subcore_barrier()

    gsems = (gsem0, gsem1)
    osems = (osem0, osem1)

    pltpu.async_copy(b_sp.at[idx_v.at[0]], rows_v.at[0], gsem0)

    def compute_chunk(c, rbuf, abuf):
        for n in range(CH):
            node = (c % (SLAB // CH)) * CH + n
            a_vecs = [a_v[node, pl.ds(16 * h, 16)] for h in range(HV)]

            def kbody(k, accs):
                out = list(accs)
                for dk in range(2):
                    r = n * K + 2 * k + dk
                    for h in range(HV):
                        row = rbuf[r, pl.ds(16 * h, 16)]
                        sv = a_vecs[h] + row
                        out[h] = out[h] + sv / (1.0 + jnp.exp(-sv))
                return tuple(out)

            accs = lax.fori_loop(
                0, K // 2, kbody,
                tuple(jnp.zeros((16,), jnp.float32) for _ in range(HV)))
            for h in range(HV):
                abuf[n, pl.ds(16 * h, 16)] = accs[h]

    def pair_fn(p, carry):
        for b in range(2):
            c = 2 * p + b

            @pl.when(jnp.logical_and(c % (SLAB // CH) == 0, c > 0))
            def _():
                pltpu.sync_copy(
                    a_hbm.at[pl.ds(base + (c // (SLAB // CH)) * SLAB, SLAB)],
                    a_v)

            @pl.when(c + 1 < NCHUNK)
            def _():
                pltpu.async_copy(b_sp.at[idx_v.at[c + 1]], rows_v.at[1 - b],
                                 gsems[1 - b])

            pltpu.make_async_copy(b_sp.at[idx_v.at[c]], rows_v.at[b],
                                  gsems[b]).wait()

            @pl.when(c >= 2)
            def _():
                pltpu.make_async_copy(acc_v.at[b], out_hbm.at[pl.ds(0, CH)],
                                      osems[b]).wait()

            compute_chunk(c, rows_v.at[b], acc_v.at[b])
            pltpu.async_copy(acc_v.at[b], out_hbm.at[pl.ds(base + c * CH, CH)],
                             osems[b])
        return carry

    lax.fori_loop(0, NCHUNK // 2, pair_fn, 0)
    pltpu.make_async_copy(acc_v.at[0], out_hbm.at[pl.ds(0, CH)], osem0).wait()
    pltpu.make_async_copy(acc_v.at[1], out_hbm.at[pl.ds(0, CH)], osem1).wait()


def kernel(x, lat_bias, knn_idx, msg_W1, msg_b1, msg_W2, msg_b2,
           upd_W1, upd_b1, upd_W2, upd_b2, ln_gamma, ln_beta):
    x2 = x[0]
    idx3 = jnp.pad(knn_idx, ((0, NPAD - N), (0, 0))).reshape(NW, NCHUNK, CH * K)

    A, B, T1 = _tc1(x2, msg_W1, msg_W1, upd_W1, upd_W1,
                    msg_b1[None], msg_b2[None], upd_b1[None])
    S = _sc_gather_silu_sum(A, B, idx3)
    out = _tc2(T1, S, x2, lat_bias[0], msg_W2, upd_W1, upd_W2, upd_b2[None],
               ln_gamma[None], ln_beta[None])
    return out[None]
